# baseline (device time: 217569 ns/iter reference)
import jax
import jax.numpy as jnp
from jax import lax
from jax.experimental import pallas as pl
from jax.experimental.pallas import tpu as pltpu

N_DEV = 4


def kernel(x, w_mat, scale_x, scale_w):
    m_per, k = x.shape
    n_total = w_mat.shape[1]
    n_per = n_total // N_DEV

    my = lax.axis_index("i")
    w_slice = lax.dynamic_slice_in_dim(w_mat, my * n_per, n_per, axis=1)
    x8 = x.astype(jnp.float8_e5m2)
    w8 = w_slice.astype(jnp.float8_e5m2)
    s = (scale_x[0] * scale_w[0]).reshape(1, 1)

    def body(x_ref, w_ref, s_ref, out_ref, comm_ref, send_sems, recv_sems):
        my_pos = lax.axis_index("i")
        left = lax.rem(my_pos + N_DEV - 1, N_DEV)
        right = lax.rem(my_pos + 1, N_DEV)

        barrier_sem = pltpu.get_barrier_semaphore()
        for nbr in (left, right):
            pl.semaphore_signal(
                barrier_sem, inc=1,
                device_id=(nbr,), device_id_type=pl.DeviceIdType.MESH,
            )
        pl.semaphore_wait(barrier_sem, 2)

        scale = s_ref[0, 0]

        def compute(src_chunk, origin):
            acc = lax.dot_general(
                src_chunk, w_ref[...],
                dimension_numbers=(((1,), (0,)), ((), ())),
                preferred_element_type=jnp.float32,
            )
            out_ref[pl.ds(origin * m_per, m_per), :] = jnp.maximum(
                acc * scale, 0.0
            )

        for h in range(N_DEV - 1):
            src = x_ref if h == 0 else comm_ref.at[h - 1]
            rdma = pltpu.make_async_remote_copy(
                src_ref=src,
                dst_ref=comm_ref.at[h],
                send_sem=send_sems.at[h],
                recv_sem=recv_sems.at[h],
                device_id=(right,),
                device_id_type=pl.DeviceIdType.MESH,
            )
            rdma.start()
            held = x_ref[...] if h == 0 else comm_ref[h - 1]
            compute(held, lax.rem(my_pos + N_DEV - h, N_DEV) if h else my_pos)
            rdma.wait()
        compute(comm_ref[N_DEV - 2], lax.rem(my_pos + 1, N_DEV))

    return pl.pallas_call(
        body,
        out_shape=jax.ShapeDtypeStruct((N_DEV * m_per, n_per), jnp.float32),
        in_specs=[
            pl.BlockSpec(memory_space=pltpu.VMEM),
            pl.BlockSpec(memory_space=pltpu.VMEM),
            pl.BlockSpec(memory_space=pltpu.SMEM),
        ],
        out_specs=pl.BlockSpec(memory_space=pltpu.VMEM),
        scratch_shapes=[
            pltpu.VMEM((N_DEV - 1, m_per, k), jnp.float8_e5m2),
            pltpu.SemaphoreType.DMA((N_DEV - 1,)),
            pltpu.SemaphoreType.DMA((N_DEV - 1,)),
        ],
        compiler_params=pltpu.CompilerParams(
            collective_id=0,
            vmem_limit_bytes=64 * 1024 * 1024,
        ),
    )(x8, w8, s)


# device time: 150100 ns/iter; 1.4495x vs baseline; 1.4495x over previous
import jax
import jax.numpy as jnp
from jax import lax
from jax.experimental import pallas as pl
from jax.experimental.pallas import tpu as pltpu

N_DEV = 4


def kernel(x, w_mat, scale_x, scale_w):
    m_per, k = x.shape
    n_total = w_mat.shape[1]
    n_per = n_total // N_DEV

    my = lax.axis_index("i")
    w_slice = lax.dynamic_slice_in_dim(w_mat, my * n_per, n_per, axis=1)
    x8 = x.astype(jnp.float8_e5m2)
    w8 = w_slice.astype(jnp.float8_e5m2)
    s = (scale_x[0] * scale_w[0]).reshape(1, 1)

    half = m_per // 2

    def body(x_ref, w_ref, s_ref, out_ref,
             cw_ref, ccw_ref, cw_send, cw_recv, ccw_send, ccw_recv):
        my_pos = lax.axis_index("i")
        left = lax.rem(my_pos + N_DEV - 1, N_DEV)
        right = lax.rem(my_pos + 1, N_DEV)

        barrier_sem = pltpu.get_barrier_semaphore()
        for nbr in (left, right):
            pl.semaphore_signal(
                barrier_sem, inc=1,
                device_id=(nbr,), device_id_type=pl.DeviceIdType.MESH,
            )
        pl.semaphore_wait(barrier_sem, 2)

        scale = s_ref[0, 0]

        def compute(src_chunk, row0, rows):
            acc = lax.dot_general(
                src_chunk, w_ref[...],
                dimension_numbers=(((1,), (0,)), ((), ())),
                preferred_element_type=jnp.float32,
            )
            out_ref[pl.ds(row0, rows), :] = jnp.maximum(acc * scale, 0.0)

        for h in range(N_DEV - 1):
            cw_src = x_ref.at[0:half] if h == 0 else cw_ref.at[h - 1]
            ccw_src = x_ref.at[half:m_per] if h == 0 else ccw_ref.at[h - 1]
            cw = pltpu.make_async_remote_copy(
                src_ref=cw_src, dst_ref=cw_ref.at[h],
                send_sem=cw_send.at[h], recv_sem=cw_recv.at[h],
                device_id=(right,), device_id_type=pl.DeviceIdType.MESH,
            )
            ccw = pltpu.make_async_remote_copy(
                src_ref=ccw_src, dst_ref=ccw_ref.at[h],
                send_sem=ccw_send.at[h], recv_sem=ccw_recv.at[h],
                device_id=(left,), device_id_type=pl.DeviceIdType.MESH,
            )
            cw.start()
            ccw.start()
            if h == 0:
                compute(x_ref[...], my_pos * m_per, m_per)
            else:
                o_cw = lax.rem(my_pos + N_DEV - h, N_DEV)
                o_ccw = lax.rem(my_pos + h, N_DEV)
                compute(cw_ref[h - 1], o_cw * m_per, half)
                compute(ccw_ref[h - 1], o_ccw * m_per + half, half)
            cw.wait()
            ccw.wait()
        o_cw = lax.rem(my_pos + 1, N_DEV)
        o_ccw = lax.rem(my_pos + N_DEV - 1, N_DEV)
        compute(cw_ref[N_DEV - 2], o_cw * m_per, half)
        compute(ccw_ref[N_DEV - 2], o_ccw * m_per + half, half)

    return pl.pallas_call(
        body,
        out_shape=jax.ShapeDtypeStruct((N_DEV * m_per, n_per), jnp.float32),
        in_specs=[
            pl.BlockSpec(memory_space=pltpu.VMEM),
            pl.BlockSpec(memory_space=pltpu.VMEM),
            pl.BlockSpec(memory_space=pltpu.SMEM),
        ],
        out_specs=pl.BlockSpec(memory_space=pltpu.VMEM),
        scratch_shapes=[
            pltpu.VMEM((N_DEV - 1, half, k), jnp.float8_e5m2),
            pltpu.VMEM((N_DEV - 1, half, k), jnp.float8_e5m2),
            pltpu.SemaphoreType.DMA((N_DEV - 1,)),
            pltpu.SemaphoreType.DMA((N_DEV - 1,)),
            pltpu.SemaphoreType.DMA((N_DEV - 1,)),
            pltpu.SemaphoreType.DMA((N_DEV - 1,)),
        ],
        compiler_params=pltpu.CompilerParams(
            collective_id=0,
            vmem_limit_bytes=64 * 1024 * 1024,
        ),
    )(x8, w8, s)


# device time: 116840 ns/iter; 1.8621x vs baseline; 1.2847x over previous
import jax
import jax.numpy as jnp
from jax import lax
from jax.experimental import pallas as pl
from jax.experimental.pallas import tpu as pltpu

N_DEV = 4


def kernel(x, w_mat, scale_x, scale_w):
    m_per, k = x.shape
    n_total = w_mat.shape[1]
    n_per = n_total // N_DEV
    half = m_per // 2
    kc = 512
    n_kc = k // kc

    s = (scale_x[0] * scale_w[0]).reshape(1, 1)

    def body(x_ref, w_hbm, s_ref, out_hbm,
             x8, wstage, w8, ostage, cw_buf, ccw_buf,
             wsem, osem, cw_send, cw_recv, ccw_send, ccw_recv):
        my_pos = lax.axis_index("i")
        left = lax.rem(my_pos + N_DEV - 1, N_DEV)
        right = lax.rem(my_pos + 1, N_DEV)
        col0 = my_pos * n_per

        wdma = [
            pltpu.make_async_copy(
                w_hbm.at[pl.ds(c * kc, kc), pl.ds(col0, n_per)],
                wstage.at[c % 2],
                wsem.at[c % 2],
            )
            for c in range(n_kc)
        ]
        wdma[0].start()

        barrier_sem = pltpu.get_barrier_semaphore()
        for nbr in (left, right):
            pl.semaphore_signal(
                barrier_sem, inc=1,
                device_id=(nbr,), device_id_type=pl.DeviceIdType.MESH,
            )
        pl.semaphore_wait(barrier_sem, 2)

        x8[...] = x_ref[...].astype(jnp.float8_e5m2)

        def hop(h):
            cw_src = x8.at[pl.ds(0, half)] if h == 0 else cw_buf.at[h - 1]
            ccw_src = x8.at[pl.ds(half, half)] if h == 0 else ccw_buf.at[h - 1]
            cw = pltpu.make_async_remote_copy(
                src_ref=cw_src, dst_ref=cw_buf.at[h],
                send_sem=cw_send.at[h], recv_sem=cw_recv.at[h],
                device_id=(right,), device_id_type=pl.DeviceIdType.MESH,
            )
            ccw = pltpu.make_async_remote_copy(
                src_ref=ccw_src, dst_ref=ccw_buf.at[h],
                send_sem=ccw_send.at[h], recv_sem=ccw_recv.at[h],
                device_id=(left,), device_id_type=pl.DeviceIdType.MESH,
            )
            cw.start()
            ccw.start()
            return cw, ccw

        cw, ccw = hop(0)

        for c in range(n_kc):
            if c + 1 < n_kc:
                wdma[c + 1].start()
            wdma[c].wait()
            w8[pl.ds(c * kc, kc), :] = wstage[c % 2].astype(jnp.float8_e5m2)

        scale = s_ref[0, 0]
        out_pending = [None, None]
        slot = [0]

        def compute(src_half, row0):
            s_i = slot[0] % 2
            slot[0] += 1
            if out_pending[s_i] is not None:
                out_pending[s_i].wait()
            acc = lax.dot_general(
                src_half, w8[...],
                dimension_numbers=(((1,), (0,)), ((), ())),
                preferred_element_type=jnp.float32,
            )
            ostage[s_i] = jnp.maximum(acc * scale, 0.0)
            h = pltpu.make_async_copy(
                ostage.at[s_i],
                out_hbm.at[pl.ds(row0, half), :],
                osem.at[s_i],
            )
            h.start()
            out_pending[s_i] = h

        compute(x8[pl.ds(0, half), :], my_pos * m_per)
        compute(x8[pl.ds(half, half), :], my_pos * m_per + half)
        cw.wait()
        ccw.wait()

        for h in range(1, N_DEV - 1):
            cw, ccw = hop(h)
            o_cw = lax.rem(my_pos + N_DEV - h, N_DEV)
            o_ccw = lax.rem(my_pos + h, N_DEV)
            compute(cw_buf[h - 1], o_cw * m_per)
            compute(ccw_buf[h - 1], o_ccw * m_per + half)
            cw.wait()
            ccw.wait()

        o_cw = lax.rem(my_pos + 1, N_DEV)
        o_ccw = lax.rem(my_pos + N_DEV - 1, N_DEV)
        compute(cw_buf[N_DEV - 2], o_cw * m_per)
        compute(ccw_buf[N_DEV - 2], o_ccw * m_per + half)
        out_pending[0].wait()
        out_pending[1].wait()

    return pl.pallas_call(
        body,
        out_shape=jax.ShapeDtypeStruct((N_DEV * m_per, n_per), jnp.float32),
        in_specs=[
            pl.BlockSpec(memory_space=pltpu.VMEM),
            pl.BlockSpec(memory_space=pl.ANY),
            pl.BlockSpec(memory_space=pltpu.SMEM),
        ],
        out_specs=pl.BlockSpec(memory_space=pl.ANY),
        scratch_shapes=[
            pltpu.VMEM((m_per, k), jnp.float8_e5m2),
            pltpu.VMEM((2, kc, n_per), jnp.float32),
            pltpu.VMEM((k, n_per), jnp.float8_e5m2),
            pltpu.VMEM((2, half, n_per), jnp.float32),
            pltpu.VMEM((N_DEV - 1, half, k), jnp.float8_e5m2),
            pltpu.VMEM((N_DEV - 1, half, k), jnp.float8_e5m2),
            pltpu.SemaphoreType.DMA((2,)),
            pltpu.SemaphoreType.DMA((2,)),
            pltpu.SemaphoreType.DMA((N_DEV - 1,)),
            pltpu.SemaphoreType.DMA((N_DEV - 1,)),
            pltpu.SemaphoreType.DMA((N_DEV - 1,)),
            pltpu.SemaphoreType.DMA((N_DEV - 1,)),
        ],
        compiler_params=pltpu.CompilerParams(
            collective_id=0,
            vmem_limit_bytes=64 * 1024 * 1024,
        ),
    )(x, w_mat, s)
